# baseline (speedup 1.0000x reference)
import functools

import jax
import jax.numpy as jnp
from jax import lax
from jax.experimental import pallas as pl
from jax.experimental.pallas import tpu as pltpu
from jax.experimental.pallas import tpu_sc as plsc

VOCAB = 100000
D = 300
DP = 304
B = 4096
HIST = 50
L = 16
NC, NS = 2, 16
NW = NC * NS
BPW = B // NW
SCALE = 1.0 / HIST
OFFS = tuple(range(0, DP, L))

_mesh = plsc.VectorSubcoreMesh(core_axis_name="c", subcore_axis_name="s")


@functools.partial(
    pl.kernel,
    mesh=_mesh,
    out_type=jax.ShapeDtypeStruct((B, DP), jnp.float32),
    scratch_types=[
        pltpu.VMEM((BPW, HIST), jnp.int32),
        pltpu.VMEM((HIST, DP), jnp.float32),
        pltpu.VMEM((HIST, DP), jnp.float32),
        pltpu.VMEM((BPW, DP), jnp.float32),
        pltpu.SemaphoreType.DMA,
        pltpu.SemaphoreType.DMA,
    ],
    compiler_params=pltpu.CompilerParams(use_tc_tiling_on_sc=False),
)
def _pool_kernel(table_hbm, idx_hbm, out_hbm, idx_v, buf_a, buf_b, acc_v,
                 sem_a, sem_b):
    wid = lax.axis_index("s") * NC + lax.axis_index("c")
    base = wid * BPW
    pltpu.sync_copy(idx_hbm.at[pl.ds(base, BPW)], idx_v)

    def start(i, buf, sem):
        pltpu.async_copy(table_hbm.at[idx_v.at[i]], buf, sem)

    def wait(buf, sem):
        pltpu.make_async_copy(table_hbm.at[pl.ds(0, HIST)], buf, sem).wait()

    def acc_elem(i, buf):
        def body(j, accs):
            return tuple(a + buf[j, pl.ds(o, L)] for a, o in zip(accs, OFFS))

        accs = lax.fori_loop(
            0, HIST, body,
            tuple(jnp.zeros((L,), jnp.float32) for _ in OFFS),
        )
        for a, o in zip(accs, OFFS):
            acc_v[i, pl.ds(o, L)] = a * SCALE

    start(0, buf_a, sem_a)
    start(1, buf_b, sem_b)

    def pipeline(t, carry):
        i0 = 2 * t
        wait(buf_a, sem_a)
        acc_elem(i0, buf_a)
        start((i0 + 2) % BPW, buf_a, sem_a)
        wait(buf_b, sem_b)
        acc_elem(i0 + 1, buf_b)
        start((i0 + 3) % BPW, buf_b, sem_b)
        return carry

    lax.fori_loop(0, BPW // 2, pipeline, 0)
    wait(buf_a, sem_a)
    wait(buf_b, sem_b)
    pltpu.sync_copy(acc_v, out_hbm.at[pl.ds(base, BPW)])


PB = 2000


def _pad_body(x_ref, o_ref):
    o_ref[:, :D] = x_ref[...]
    o_ref[:, D:] = jnp.zeros((PB, DP - D), jnp.float32)


_pad_table = pl.pallas_call(
    _pad_body,
    grid=(VOCAB // PB,),
    in_specs=[pl.BlockSpec((PB, D), lambda i: (i, 0))],
    out_specs=pl.BlockSpec((PB, DP), lambda i: (i, 0)),
    out_shape=jax.ShapeDtypeStruct((VOCAB, DP), jnp.float32),
)


MB = 512


def _mm_body(x_ref, w_ref, b_ref, o_ref):
    o_ref[...] = lax.dot_general(
        x_ref[...], w_ref[...], (((1,), (1,)), ((), ())),
        preferred_element_type=jnp.float32,
    ) + b_ref[...]


_matmul = pl.pallas_call(
    _mm_body,
    grid=(B // MB,),
    in_specs=[
        pl.BlockSpec((MB, DP), lambda i: (i, 0)),
        pl.BlockSpec((D, DP), lambda i: (0, 0)),
        pl.BlockSpec((1, D), lambda i: (0, 0)),
    ],
    out_specs=pl.BlockSpec((MB, D), lambda i: (i, 0)),
    out_shape=jax.ShapeDtypeStruct((B, D), jnp.float32),
)


def kernel(table, W, b, indices):
    idx = indices.astype(jnp.int32)
    one = (indices[0, 0] * 0 + 1).astype(jnp.float32)
    table_p = jnp.pad(table, ((0, 0), (0, DP - D))) * one
    w_p = jnp.pad(W, ((0, 0), (0, DP - D)))
    pooled = _pool_kernel(table_p, idx)
    return _matmul(pooled, w_p, b.reshape(1, D))


# speedup vs baseline: 1.6409x; 1.6409x over previous
import functools

import jax
import jax.numpy as jnp
from jax import lax
from jax.experimental import pallas as pl
from jax.experimental.pallas import tpu as pltpu
from jax.experimental.pallas import tpu_sc as plsc

VOCAB = 100000
D = 300
DP = 304
B = 4096
HIST = 50
L = 16
NC, NS = 2, 16
NW = NC * NS
BPW = B // NW
SCALE = 1.0 / HIST
OFFS = tuple(range(0, DP, L))

_mesh = plsc.VectorSubcoreMesh(core_axis_name="c", subcore_axis_name="s")


@functools.partial(
    pl.kernel,
    mesh=_mesh,
    out_type=jax.ShapeDtypeStruct((B, DP), jnp.float32),
    scratch_types=[
        pltpu.VMEM((BPW, HIST), jnp.int32),
        pltpu.VMEM((HIST, DP), jnp.float32),
        pltpu.VMEM((HIST, DP), jnp.float32),
        pltpu.VMEM((BPW, DP), jnp.float32),
        pltpu.SemaphoreType.DMA,
        pltpu.SemaphoreType.DMA,
    ],
    compiler_params=pltpu.CompilerParams(use_tc_tiling_on_sc=False),
)
def _pool_kernel(table_hbm, idx_hbm, out_hbm, idx_v, buf_a, buf_b, acc_v,
                 sem_a, sem_b):
    wid = lax.axis_index("s") * NC + lax.axis_index("c")
    base = wid * BPW
    pltpu.sync_copy(idx_hbm.at[pl.ds(base, BPW)], idx_v)

    def start(i, buf, sem):
        pltpu.async_copy(table_hbm.at[idx_v.at[i]], buf, sem)

    def wait(buf, sem):
        pltpu.make_async_copy(table_hbm.at[pl.ds(0, HIST)], buf, sem).wait()

    def acc_elem(i, buf):
        def body(j, accs):
            return tuple(a + buf[j, pl.ds(o, L)] for a, o in zip(accs, OFFS))

        accs = lax.fori_loop(
            0, HIST, body,
            tuple(jnp.zeros((L,), jnp.float32) for _ in OFFS),
        )
        for a, o in zip(accs, OFFS):
            acc_v[i, pl.ds(o, L)] = a * SCALE

    start(0, buf_a, sem_a)
    start(1, buf_b, sem_b)

    def pipeline(t, carry):
        i0 = 2 * t
        wait(buf_a, sem_a)
        acc_elem(i0, buf_a)
        start((i0 + 2) % BPW, buf_a, sem_a)
        wait(buf_b, sem_b)
        acc_elem(i0 + 1, buf_b)
        start((i0 + 3) % BPW, buf_b, sem_b)
        return carry

    lax.fori_loop(0, BPW // 2, pipeline, 0)
    wait(buf_a, sem_a)
    wait(buf_b, sem_b)
    pltpu.sync_copy(acc_v, out_hbm.at[pl.ds(base, BPW)])


RB = 2000


def _pad_body(x_ref, o_ref):
    o_ref[...] = jnp.pad(x_ref[...], ((0, 0), (0, DP - D)))


_pad_tc = pl.pallas_call(
    _pad_body,
    grid=(VOCAB // RB,),
    in_specs=[pl.BlockSpec((RB, D), lambda i: (i, 0))],
    out_specs=pl.BlockSpec((RB, DP), lambda i: (i, 0)),
    out_shape=jax.ShapeDtypeStruct((VOCAB, DP), jnp.float32),
)


MB = 512


def _mm_body(x_ref, w_ref, b_ref, o_ref):
    o_ref[...] = lax.dot_general(
        x_ref[...], w_ref[...], (((1,), (1,)), ((), ())),
        preferred_element_type=jnp.float32,
    ) + b_ref[...]


_matmul = pl.pallas_call(
    _mm_body,
    grid=(B // MB,),
    in_specs=[
        pl.BlockSpec((MB, DP), lambda i: (i, 0)),
        pl.BlockSpec((D, DP), lambda i: (0, 0)),
        pl.BlockSpec((1, D), lambda i: (0, 0)),
    ],
    out_specs=pl.BlockSpec((MB, D), lambda i: (i, 0)),
    out_shape=jax.ShapeDtypeStruct((B, D), jnp.float32),
)


def kernel(table, W, b, indices):
    idx = indices.astype(jnp.int32)
    table_p = _pad_tc(table)
    w_p = jnp.pad(W, ((0, 0), (0, DP - D)))
    pooled = _pool_kernel(table_p, idx)
    return _matmul(pooled, w_p, b.reshape(1, D))


# speedup vs baseline: 1.6507x; 1.0060x over previous
import functools

import jax
import jax.numpy as jnp
from jax import lax
from jax.experimental import pallas as pl
from jax.experimental.pallas import tpu as pltpu
from jax.experimental.pallas import tpu_sc as plsc

VOCAB = 100000
D = 300
DP = 304
B = 4096
HIST = 50
L = 16
NC, NS = 2, 16
NW = NC * NS
BPW = B // NW
SCALE = 1.0 / HIST
OFFS = tuple(range(0, DP, L))

_mesh = plsc.VectorSubcoreMesh(core_axis_name="c", subcore_axis_name="s")


@functools.partial(
    pl.kernel,
    mesh=_mesh,
    out_type=jax.ShapeDtypeStruct((B, DP), jnp.float32),
    scratch_types=[
        pltpu.VMEM((BPW, HIST), jnp.int32),
        pltpu.VMEM((HIST, DP), jnp.float32),
        pltpu.VMEM((HIST, DP), jnp.float32),
        pltpu.VMEM((BPW, DP), jnp.float32),
        pltpu.SemaphoreType.DMA,
        pltpu.SemaphoreType.DMA,
    ],
    compiler_params=pltpu.CompilerParams(use_tc_tiling_on_sc=False),
)
def _pool_kernel(table_hbm, idx_hbm, out_hbm, idx_v, buf_a, buf_b, acc_v,
                 sem_a, sem_b):
    wid = lax.axis_index("s") * NC + lax.axis_index("c")
    base = wid * BPW
    pltpu.sync_copy(idx_hbm.at[pl.ds(base, BPW)], idx_v)

    def start(i, buf, sem):
        pltpu.async_copy(table_hbm.at[idx_v.at[i]], buf, sem)

    def wait(buf, sem):
        pltpu.make_async_copy(table_hbm.at[pl.ds(0, HIST)], buf, sem).wait()

    def acc_elem(i, buf):
        def body(j, accs):
            return tuple(a + buf[j, pl.ds(o, L)] for a, o in zip(accs, OFFS))

        accs = lax.fori_loop(
            0, HIST, body,
            tuple(jnp.zeros((L,), jnp.float32) for _ in OFFS),
        )
        for a, o in zip(accs, OFFS):
            acc_v[i, pl.ds(o, L)] = a * SCALE

    start(0, buf_a, sem_a)
    start(1, buf_b, sem_b)

    def pipeline(t, carry):
        i0 = 2 * t
        wait(buf_a, sem_a)
        acc_elem(i0, buf_a)
        start((i0 + 2) % BPW, buf_a, sem_a)
        wait(buf_b, sem_b)
        acc_elem(i0 + 1, buf_b)
        start((i0 + 3) % BPW, buf_b, sem_b)
        return carry

    lax.fori_loop(0, BPW // 2, pipeline, 0)
    wait(buf_a, sem_a)
    wait(buf_b, sem_b)
    pltpu.sync_copy(acc_v, out_hbm.at[pl.ds(base, BPW)])


RB = 4000


def _pad_body(x_ref, o_ref):
    o_ref[...] = jnp.pad(x_ref[...], ((0, 0), (0, DP - D)))


_pad_tc = pl.pallas_call(
    _pad_body,
    grid=(VOCAB // RB,),
    in_specs=[pl.BlockSpec((RB, D), lambda i: (i, 0))],
    out_specs=pl.BlockSpec((RB, DP), lambda i: (i, 0)),
    out_shape=jax.ShapeDtypeStruct((VOCAB, DP), jnp.float32),
    compiler_params=pltpu.CompilerParams(
        dimension_semantics=("parallel",)),
)


MB = 512


def _mm_body(x_ref, w_ref, b_ref, o_ref):
    o_ref[...] = lax.dot_general(
        x_ref[...], w_ref[...], (((1,), (1,)), ((), ())),
        preferred_element_type=jnp.float32,
    ) + b_ref[...]


_matmul = pl.pallas_call(
    _mm_body,
    grid=(B // MB,),
    in_specs=[
        pl.BlockSpec((MB, DP), lambda i: (i, 0)),
        pl.BlockSpec((D, DP), lambda i: (0, 0)),
        pl.BlockSpec((1, D), lambda i: (0, 0)),
    ],
    out_specs=pl.BlockSpec((MB, D), lambda i: (i, 0)),
    out_shape=jax.ShapeDtypeStruct((B, D), jnp.float32),
)


def kernel(table, W, b, indices):
    idx = indices.astype(jnp.int32)
    table_p = _pad_tc(table)
    w_p = jnp.pad(W, ((0, 0), (0, DP - D)))
    pooled = _pool_kernel(table_p, idx)
    return _matmul(pooled, w_p, b.reshape(1, D))
